# Initial kernel scaffold; baseline (speedup 1.0000x reference)
import jax
import jax.numpy as jnp
from jax.experimental import pallas as pl


def kernel(X, W):
    raise NotImplementedError("write your pallas kernel here")


# speedup vs baseline: 1.1015x; 1.1015x over previous
import functools

import jax
import jax.numpy as jnp
from jax import lax
from jax.experimental import pallas as pl
from jax.experimental.pallas import tpu as pltpu
from jax.experimental.pallas import tpu_sc as plsc

NUM_EMB = 1_000_000
DIM = 32
BATCH = 16384
FIELDS = 100
TOTAL = BATCH * FIELDS

NUM_CORES = 2
NUM_SUBCORES = 16
NUM_WORKERS = NUM_CORES * NUM_SUBCORES
PER_WORKER = TOTAL // NUM_WORKERS
CHUNK = 1024
NUM_CHUNKS = PER_WORKER // CHUNK


def _build():
    mesh = plsc.VectorSubcoreMesh(core_axis_name="c", subcore_axis_name="s")

    @functools.partial(
        pl.kernel,
        mesh=mesh,
        out_type=jax.ShapeDtypeStruct((TOTAL, DIM), jnp.float32),
        scratch_types=[
            pltpu.VMEM((CHUNK,), jnp.int32),
            pltpu.VMEM((CHUNK, DIM), jnp.float32),
            pltpu.SemaphoreType.DMA,
        ],
        compiler_params=pltpu.CompilerParams(use_tc_tiling_on_sc=False),
    )
    def emb_kernel(idx_hbm, table_hbm, out_hbm, idx_v, rows_v, sem):
        wid = lax.axis_index("s") * NUM_CORES + lax.axis_index("c")
        base = wid * PER_WORKER

        def body(g, carry):
            off = base + g * CHUNK
            pltpu.sync_copy(idx_hbm.at[pl.ds(off, CHUNK)], idx_v)
            pltpu.async_copy(table_hbm.at[idx_v], rows_v, sem).wait()
            pltpu.sync_copy(rows_v, out_hbm.at[pl.ds(off, CHUNK)])
            return carry

        lax.fori_loop(0, NUM_CHUNKS, body, 0)

    return emb_kernel


_emb_kernel = _build()


def kernel(X, W):
    idx = X.reshape(TOTAL).astype(jnp.int32)
    out = _emb_kernel(idx, W)
    return out.reshape(BATCH, FIELDS, DIM)


# speedup vs baseline: 1.1118x; 1.0093x over previous
import functools

import jax
import jax.numpy as jnp
from jax import lax
from jax.experimental import pallas as pl
from jax.experimental.pallas import tpu as pltpu
from jax.experimental.pallas import tpu_sc as plsc

NUM_EMB = 1_000_000
DIM = 32
BATCH = 16384
FIELDS = 100
TOTAL = BATCH * FIELDS

NUM_CORES = 2
NUM_SUBCORES = 16
NUM_WORKERS = NUM_CORES * NUM_SUBCORES
PER_WORKER = TOTAL // NUM_WORKERS
CHUNK = 1600
NUM_CHUNKS = PER_WORKER // CHUNK
NSLOT = 2


def _build():
    mesh = plsc.VectorSubcoreMesh(core_axis_name="c", subcore_axis_name="s")

    @functools.partial(
        pl.kernel,
        mesh=mesh,
        out_type=jax.ShapeDtypeStruct((TOTAL, DIM), jnp.float32),
        scratch_types=[
            [pltpu.VMEM((CHUNK,), jnp.int32) for _ in range(NSLOT)],
            [pltpu.VMEM((CHUNK, DIM), jnp.float32) for _ in range(NSLOT)],
            [pltpu.SemaphoreType.DMA for _ in range(NSLOT)],
            [pltpu.SemaphoreType.DMA for _ in range(NSLOT)],
            [pltpu.SemaphoreType.DMA for _ in range(NSLOT)],
        ],
        compiler_params=pltpu.CompilerParams(use_tc_tiling_on_sc=False),
    )
    def emb_kernel(idx_hbm, table_hbm, out_hbm, idx_v, rows_v, idx_sem,
                   gat_sem, out_sem):
        wid = lax.axis_index("s") * NUM_CORES + lax.axis_index("c")
        base = wid * PER_WORKER

        def issue_idx(i, b):
            pltpu.make_async_copy(
                idx_hbm.at[pl.ds(base + i * CHUNK, CHUNK)], idx_v[b],
                idx_sem[b]).start()

        def wait_idx(b):
            pltpu.make_async_copy(
                idx_hbm.at[pl.ds(base, CHUNK)], idx_v[b], idx_sem[b]).wait()

        def issue_gather(b):
            pltpu.make_async_copy(
                table_hbm.at[idx_v[b]], rows_v[b], gat_sem[b]).start()

        def wait_gather(b):
            pltpu.make_async_copy(
                table_hbm.at[idx_v[b]], rows_v[b], gat_sem[b]).wait()

        def issue_out(i, b):
            pltpu.make_async_copy(
                rows_v[b], out_hbm.at[pl.ds(base + i * CHUNK, CHUNK)],
                out_sem[b]).start()

        def wait_out(b):
            pltpu.make_async_copy(
                rows_v[b], out_hbm.at[pl.ds(base, CHUNK)], out_sem[b]).wait()

        for b in range(NSLOT):
            issue_idx(b, b)
        for b in range(NSLOT):
            wait_idx(b)
            issue_gather(b)
            wait_gather(b)
            issue_out(b, b)
            issue_idx(b + NSLOT, b)

        def body(gg, carry):
            for b in range(NSLOT):
                i = gg * NSLOT + b
                wait_idx(b)
                wait_out(b)
                issue_gather(b)
                wait_gather(b)
                issue_out(i, b)
                issue_idx(i + NSLOT, b)
            return carry

        lax.fori_loop(1, NUM_CHUNKS // NSLOT - 1, body, 0)

        for b in range(NSLOT):
            i = NUM_CHUNKS - NSLOT + b
            wait_idx(b)
            wait_out(b)
            issue_gather(b)
            wait_gather(b)
            issue_out(i, b)
        for b in range(NSLOT):
            wait_out(b)

    return emb_kernel


_emb_kernel = _build()


def kernel(X, W):
    idx = X.reshape(TOTAL).astype(jnp.int32)
    out = _emb_kernel(idx, W)
    return out.reshape(BATCH, FIELDS, DIM)


# speedup vs baseline: 3.2187x; 2.8950x over previous
import functools

import jax
import jax.numpy as jnp
from jax import lax
from jax.experimental import pallas as pl
from jax.experimental.pallas import tpu as pltpu
from jax.experimental.pallas import tpu_sc as plsc

NUM_EMB = 1_000_000
DIM = 32
BATCH = 16384
FIELDS = 100
TOTAL = BATCH * FIELDS

NUM_CORES = 2
NUM_SUBCORES = 16
NUM_WORKERS = NUM_CORES * NUM_SUBCORES
CHUNK = BATCH // NUM_WORKERS
NSLOT = 2
LANES = 16


def _build():
    mesh = plsc.VectorSubcoreMesh(core_axis_name="c", subcore_axis_name="s")

    @functools.partial(
        pl.kernel,
        mesh=mesh,
        out_type=jax.ShapeDtypeStruct((FIELDS, DIM, BATCH), jnp.float32),
        scratch_types=[
            [pltpu.VMEM((CHUNK,), jnp.int32) for _ in range(NSLOT)],
            [pltpu.VMEM((CHUNK, DIM), jnp.float32) for _ in range(NSLOT)],
            [pltpu.VMEM((DIM, CHUNK), jnp.float32) for _ in range(NSLOT)],
            [pltpu.SemaphoreType.DMA for _ in range(NSLOT)],
            [pltpu.SemaphoreType.DMA for _ in range(NSLOT)],
            [pltpu.SemaphoreType.DMA for _ in range(NSLOT)],
        ],
        compiler_params=pltpu.CompilerParams(
            use_tc_tiling_on_sc=False, needs_layout_passes=False),
    )
    def emb_kernel(idx_hbm, table_hbm, out_hbm, idx_v, rows_v, trans_v,
                   idx_sem, gat_sem, out_sem):
        wid = lax.axis_index("s") * NUM_CORES + lax.axis_index("c")
        bbase = wid * CHUNK

        def issue_idx(f, b):
            pltpu.make_async_copy(
                idx_hbm.at[pl.ds(f * BATCH + bbase, CHUNK)], idx_v[b],
                idx_sem[b]).start()

        def wait_idx(b):
            pltpu.make_async_copy(
                idx_hbm.at[pl.ds(bbase, CHUNK)], idx_v[b], idx_sem[b]).wait()

        def issue_gather(b):
            pltpu.make_async_copy(
                table_hbm.at[idx_v[b]], rows_v[b], gat_sem[b]).start()

        def wait_gather(b):
            pltpu.make_async_copy(
                table_hbm.at[idx_v[b]], rows_v[b], gat_sem[b]).wait()

        def issue_out(f, b):
            pltpu.make_async_copy(
                trans_v[b], out_hbm.at[f, :, pl.ds(bbase, CHUNK)],
                out_sem[b]).start()

        def wait_out(b):
            pltpu.make_async_copy(
                trans_v[b], out_hbm.at[0, :, pl.ds(bbase, CHUNK)],
                out_sem[b]).wait()

        lane_iota = lax.iota(jnp.int32, LANES)

        def transpose(b):
            def d_body(d, carry):
                for cg in range(CHUNK // LANES):
                    row_idx = cg * LANES + lane_iota
                    col_idx = jnp.full((LANES,), d, jnp.int32)
                    vec = plsc.load_gather(rows_v[b], [row_idx, col_idx])
                    trans_v[b][d, pl.ds(cg * LANES, LANES)] = vec
                return carry

            lax.fori_loop(0, DIM, d_body, 0)

        for b in range(NSLOT):
            issue_idx(b, b)
        for b in range(NSLOT):
            wait_idx(b)
            issue_gather(b)
            wait_gather(b)
            transpose(b)
            issue_out(b, b)
            issue_idx(b + NSLOT, b)

        def body(gg, carry):
            for b in range(NSLOT):
                f = gg * NSLOT + b
                wait_idx(b)
                wait_out(b)
                issue_gather(b)
                wait_gather(b)
                transpose(b)
                issue_out(f, b)
                issue_idx(f + NSLOT, b)
            return carry

        lax.fori_loop(1, FIELDS // NSLOT - 1, body, 0)

        for b in range(NSLOT):
            f = FIELDS - NSLOT + b
            wait_idx(b)
            wait_out(b)
            issue_gather(b)
            wait_gather(b)
            transpose(b)
            issue_out(f, b)
        for b in range(NSLOT):
            wait_out(b)

    return emb_kernel


_emb_kernel = _build()


def kernel(X, W):
    idx = X.T.reshape(TOTAL).astype(jnp.int32)
    out = _emb_kernel(idx, W)
    return out.transpose(2, 0, 1)


# speedup vs baseline: 3.3816x; 1.0506x over previous
import functools

import jax
import jax.numpy as jnp
from jax import lax
from jax.experimental import pallas as pl
from jax.experimental.pallas import tpu as pltpu
from jax.experimental.pallas import tpu_sc as plsc

NUM_EMB = 1_000_000
DIM = 32
BATCH = 16384
FIELDS = 100
TOTAL = BATCH * FIELDS

NUM_CORES = 2
NUM_SUBCORES = 16
NUM_WORKERS = NUM_CORES * NUM_SUBCORES
CHUNK = BATCH // NUM_WORKERS
NSLOT = 2
LANES = 16


def _build():
    mesh = plsc.VectorSubcoreMesh(core_axis_name="c", subcore_axis_name="s")

    @functools.partial(
        pl.kernel,
        mesh=mesh,
        out_type=jax.ShapeDtypeStruct((FIELDS, DIM, BATCH), jnp.float32),
        scratch_types=[
            [pltpu.VMEM((CHUNK,), jnp.int32) for _ in range(NSLOT)],
            [pltpu.VMEM((CHUNK, DIM), jnp.float32) for _ in range(NSLOT)],
            [pltpu.VMEM((DIM, CHUNK), jnp.float32) for _ in range(NSLOT)],
            [pltpu.SemaphoreType.DMA for _ in range(NSLOT)],
            [pltpu.SemaphoreType.DMA for _ in range(NSLOT)],
            [pltpu.SemaphoreType.DMA for _ in range(NSLOT)],
        ],
        compiler_params=pltpu.CompilerParams(
            use_tc_tiling_on_sc=False, needs_layout_passes=False),
    )
    def emb_kernel(idx_hbm, table_hbm, out_hbm, idx_v, rows_v, trans_v,
                   idx_sem, gat_sem, out_sem):
        wid = lax.axis_index("s") * NUM_CORES + lax.axis_index("c")
        bbase = wid * CHUNK

        def issue_idx(f, b):
            pltpu.make_async_copy(
                idx_hbm.at[pl.ds(f * BATCH + bbase, CHUNK)], idx_v[b],
                idx_sem[b]).start()

        def wait_idx(b):
            pltpu.make_async_copy(
                idx_hbm.at[pl.ds(bbase, CHUNK)], idx_v[b], idx_sem[b]).wait()

        def issue_gather(b):
            pltpu.make_async_copy(
                table_hbm.at[idx_v[b]], rows_v[b], gat_sem[b]).start()

        def wait_gather(b):
            pltpu.make_async_copy(
                table_hbm.at[idx_v[b]], rows_v[b], gat_sem[b]).wait()

        def issue_out(f, b):
            pltpu.make_async_copy(
                trans_v[b], out_hbm.at[f, :, pl.ds(bbase, CHUNK)],
                out_sem[b]).start()

        def wait_out(b):
            pltpu.make_async_copy(
                trans_v[b], out_hbm.at[0, :, pl.ds(bbase, CHUNK)],
                out_sem[b]).wait()

        lane_iota = lax.iota(jnp.int32, LANES)

        def transpose(b):
            def cg_body(cg, carry):
                row_idx = cg * LANES + lane_iota
                base = cg * LANES
                for d in range(DIM):
                    col_idx = jnp.full((LANES,), d, jnp.int32)
                    vec = plsc.load_gather(rows_v[b], [row_idx, col_idx])
                    trans_v[b][d, pl.ds(base, LANES)] = vec
                return carry

            lax.fori_loop(0, CHUNK // LANES, cg_body, 0)

        issue_idx(0, 0)
        issue_idx(1, 1)
        wait_idx(0)
        issue_gather(0)
        wait_gather(0)
        wait_idx(1)
        issue_gather(1)
        transpose(0)
        issue_out(0, 0)
        issue_idx(2, 0)
        wait_gather(1)
        wait_idx(0)
        issue_gather(0)
        transpose(1)
        issue_out(1, 1)
        issue_idx(3, 1)

        def body(gg, carry):
            for b in range(NSLOT):
                f = gg * NSLOT + b
                b2 = 1 - b
                wait_gather(b)
                wait_idx(b2)
                issue_gather(b2)
                wait_out(b)
                transpose(b)
                issue_out(f, b)
                issue_idx(f + NSLOT, b)
            return carry

        lax.fori_loop(1, FIELDS // NSLOT - 1, body, 0)

        wait_gather(0)
        wait_idx(1)
        issue_gather(1)
        wait_out(0)
        transpose(0)
        issue_out(FIELDS - 2, 0)
        wait_gather(1)
        wait_out(1)
        transpose(1)
        issue_out(FIELDS - 1, 1)
        for b in range(NSLOT):
            wait_out(b)

    return emb_kernel


_emb_kernel = _build()


def kernel(X, W):
    idx = X.T.reshape(TOTAL).astype(jnp.int32)
    out = _emb_kernel(idx, W)
    return out.transpose(2, 0, 1)


# speedup vs baseline: 4.5731x; 1.3524x over previous
import functools

import jax
import jax.numpy as jnp
from jax import lax
from jax.experimental import pallas as pl
from jax.experimental.pallas import tpu as pltpu
from jax.experimental.pallas import tpu_sc as plsc

NUM_EMB = 1_000_000
DIM = 32
BATCH = 16384
FIELDS = 100
TOTAL = BATCH * FIELDS

NUM_CORES = 2
NUM_SUBCORES = 16
NUM_WORKERS = NUM_CORES * NUM_SUBCORES
CHUNK = BATCH // NUM_WORKERS
NSLOT = 2
LANES = 16


def _build():
    mesh = plsc.VectorSubcoreMesh(core_axis_name="c", subcore_axis_name="s")

    @functools.partial(
        pl.kernel,
        mesh=mesh,
        out_type=jax.ShapeDtypeStruct((FIELDS, DIM, BATCH), jnp.float32),
        scratch_types=[
            [pltpu.VMEM((CHUNK,), jnp.int32) for _ in range(NSLOT)],
            [pltpu.VMEM((CHUNK, DIM), jnp.float32) for _ in range(NSLOT)],
            [pltpu.VMEM((DIM, CHUNK), jnp.float32) for _ in range(NSLOT)],
            [pltpu.SemaphoreType.DMA for _ in range(NSLOT)],
            [pltpu.SemaphoreType.DMA for _ in range(NSLOT)],
            [pltpu.SemaphoreType.DMA for _ in range(NSLOT)],
        ],
        compiler_params=pltpu.CompilerParams(
            use_tc_tiling_on_sc=False, needs_layout_passes=False),
    )
    def emb_kernel(idx_hbm, table_hbm, out_hbm, idx_v, rows_v, trans_v,
                   idx_sem, gat_sem, out_sem):
        wid = lax.axis_index("s") * NUM_CORES + lax.axis_index("c")
        bbase = wid * CHUNK

        def issue_idx(f, b):
            pltpu.make_async_copy(
                idx_hbm.at[pl.ds(f * BATCH + bbase, CHUNK)], idx_v[b],
                idx_sem[b]).start()

        def wait_idx(b):
            pltpu.make_async_copy(
                idx_hbm.at[pl.ds(bbase, CHUNK)], idx_v[b], idx_sem[b]).wait()

        def issue_gather(b):
            pltpu.make_async_copy(
                table_hbm.at[idx_v[b]], rows_v[b], gat_sem[b]).start()

        def wait_gather(b):
            pltpu.make_async_copy(
                table_hbm.at[idx_v[b]], rows_v[b], gat_sem[b]).wait()

        def issue_out(f, b):
            pltpu.make_async_copy(
                trans_v[b], out_hbm.at[f, :, pl.ds(bbase, CHUNK)],
                out_sem[b]).start()

        def wait_out(b):
            pltpu.make_async_copy(
                trans_v[b], out_hbm.at[0, :, pl.ds(bbase, CHUNK)],
                out_sem[b]).wait()

        lane_iota = lax.iota(jnp.int32, LANES)

        def transpose(b):
            @plsc.parallel_loop(0, CHUNK, unroll=8)
            def j_body(j):
                col_idx = jnp.full((LANES,), j, jnp.int32)
                for dg in range(DIM // LANES):
                    vec = rows_v[b][j, pl.ds(dg * LANES, LANES)]
                    plsc.store_scatter(
                        trans_v[b], [dg * LANES + lane_iota, col_idx], vec)

        issue_idx(0, 0)
        issue_idx(1, 1)
        wait_idx(0)
        issue_gather(0)
        wait_gather(0)
        wait_idx(1)
        issue_gather(1)
        transpose(0)
        issue_out(0, 0)
        issue_idx(2, 0)
        wait_gather(1)
        wait_idx(0)
        issue_gather(0)
        transpose(1)
        issue_out(1, 1)
        issue_idx(3, 1)

        def body(gg, carry):
            for b in range(NSLOT):
                f = gg * NSLOT + b
                b2 = 1 - b
                wait_gather(b)
                wait_idx(b2)
                issue_gather(b2)
                wait_out(b)
                transpose(b)
                issue_out(f, b)
                issue_idx(f + NSLOT, b)
            return carry

        lax.fori_loop(1, FIELDS // NSLOT - 1, body, 0)

        wait_gather(0)
        wait_idx(1)
        issue_gather(1)
        wait_out(0)
        transpose(0)
        issue_out(FIELDS - 2, 0)
        wait_gather(1)
        wait_out(1)
        transpose(1)
        issue_out(FIELDS - 1, 1)
        for b in range(NSLOT):
            wait_out(b)

    return emb_kernel


_emb_kernel = _build()


def kernel(X, W):
    idx = X.T.reshape(TOTAL).astype(jnp.int32)
    out = _emb_kernel(idx, W)
    return out.transpose(2, 0, 1)
